# Initial kernel scaffold; baseline (speedup 1.0000x reference)
import functools

import jax
import jax.numpy as jnp
import numpy as np
from jax import lax
from jax.experimental import pallas as pl
from jax.experimental.pallas import tpu as pltpu
from jax.experimental.pallas import tpu_sc as plsc

N = 10000
E = 320000
D = 128
NC = 2
NS = 16
NW = NC * NS
CHUNK = 128
ROWS_PAD = 10240
ROW_TRASH = N

NODE_SIZES = [121, 11, 11, 4, 7, 9, 7]
EDGE_SIZES = [7, 3, 3, 6]
NPOS = sum(EDGE_SIZES)
WCNT = 32
NCOMBO = 7 * 3 * 3 * 6


def _sc_gather_scatter(chunks: int, d: int):
  mesh = plsc.VectorSubcoreMesh(core_axis_name="c", subcore_axis_name="s")
  rows_per_sub = ROWS_PAD // NS
  n_out_blk = rows_per_sub // CHUNK

  @functools.partial(
      pl.kernel,
      out_type=jax.ShapeDtypeStruct((NC, ROWS_PAD, d), jnp.float32),
      mesh=mesh,
      scratch_types=[
          pltpu.VMEM((chunks, CHUNK), jnp.int32),
          pltpu.VMEM((chunks, CHUNK), jnp.int32),
          pltpu.VMEM((CHUNK, d), jnp.float32),
          pltpu.VMEM_SHARED((ROWS_PAD, d), jnp.float32),
          pltpu.SemaphoreType.DMA,
      ],
  )
  def k(table_hbm, gidx_hbm, sidx_hbm, zeros_hbm, out_hbm,
        gi_v, si_v, rows_v, agg_sh, sem):
    c = lax.axis_index("c")
    s = lax.axis_index("s")
    wid = c * NS + s
    pltpu.sync_copy(gidx_hbm.at[pl.ds(wid * chunks, chunks)], gi_v)
    pltpu.sync_copy(sidx_hbm.at[pl.ds(wid * chunks, chunks)], si_v)
    pltpu.sync_copy(zeros_hbm.at[pl.ds(s * rows_per_sub, rows_per_sub)],
                    agg_sh.at[pl.ds(s * rows_per_sub, rows_per_sub)])
    plsc.subcore_barrier()

    def body(j, carry):
      pltpu.async_copy(table_hbm.at[gi_v.at[j]], rows_v, sem).wait()
      pltpu.sync_copy(rows_v, agg_sh.at[si_v.at[j]], add=True)
      return carry

    lax.fori_loop(0, chunks, body, 0, unroll=False)
    plsc.subcore_barrier()
    for b in range(n_out_blk):
      r0 = s * rows_per_sub + b * CHUNK
      pltpu.sync_copy(agg_sh.at[pl.ds(r0, CHUNK)], rows_v)
      pltpu.sync_copy(rows_v, out_hbm.at[c].at[pl.ds(r0, CHUNK)])

  return k


def _pad_idx(a, pad_val, total):
  a = a.astype(jnp.int32)
  return jnp.concatenate(
      [a, jnp.full((total - a.shape[0],), pad_val, jnp.int32)]
  ).reshape(-1, CHUNK)


def _dense_update(elu: bool):
  def body(a0, a1, h, c0, c1, selfrow, ep, w1, b1, w2, b2, gamma, beta, out):
    agg = a0[...] + a1[...] + h[...]
    cnt = c0[...] + c1[...] + selfrow[...]
    agg = agg + jnp.dot(cnt, ep[...], preferred_element_type=jnp.float32)
    z1 = jnp.maximum(
        jnp.dot(agg, w1[...], preferred_element_type=jnp.float32) + b1[...],
        0.0)
    z = jnp.dot(z1, w2[...], preferred_element_type=jnp.float32) + b2[...]
    mean = jnp.sum(z, axis=0, keepdims=True) * (1.0 / N)
    zc = z - mean
    var = jnp.sum(zc * zc, axis=0, keepdims=True) * (1.0 / N)
    zn = zc * lax.rsqrt(var + 1e-5) * gamma[...] + beta[...]
    out[...] = jnp.where(zn > 0, zn, jnp.expm1(zn)) if elu else zn
  return body


def _add2(a0, a1, out):
  out[...] = a0[...] + a1[...]


def kernel(x, edge_index, edge_attr, params):
  f32 = jnp.float32
  src = edge_index[0].astype(jnp.int32)
  dst = edge_index[1].astype(jnp.int32)
  ea = edge_attr.astype(jnp.int32)

  oh_tab = np.zeros((NCOMBO + 6, WCNT), np.float32)
  offs = np.cumsum([0] + EDGE_SIZES[:-1])
  for cmb in range(NCOMBO):
    a0v, r = divmod(cmb, 54)
    a1v, r = divmod(r, 18)
    a2v, a3v = divmod(r, 6)
    for f, v in enumerate((a0v, a1v, a2v, a3v)):
      oh_tab[cmb, offs[f] + v] += 1.0
  selfrow = np.zeros((1, WCNT), np.float32)
  selfrow[0, [4, 7, 10, 13]] = 1.0

  combo = ((ea[:, 0] * 3 + ea[:, 1]) * 3 + ea[:, 2]) * 6 + ea[:, 3]

  ep_edges = NW * 80 * CHUNK
  g_edge = _pad_idx(src, 0, ep_edges)
  s_edge = _pad_idx(dst, ROW_TRASH, ep_edges)
  g_cnt = _pad_idx(combo, NCOMBO, ep_edges)

  node_tab = jnp.concatenate([t for t in params["node_emb"]], axis=0)
  noffs = jnp.asarray(np.cumsum([0] + NODE_SIZES[:-1]), jnp.int32)
  h0_g = (x.astype(jnp.int32) + noffs[None, :]).reshape(-1)
  h0_s = jnp.repeat(jnp.arange(N, dtype=jnp.int32), 7)
  ep_h0 = NW * 18 * CHUNK
  g_h0 = _pad_idx(h0_g, 0, ep_h0)
  s_h0 = _pad_idx(h0_s, ROW_TRASH, ep_h0)

  zeros_d = jnp.zeros((ROWS_PAD, D), f32)
  zeros_w = jnp.zeros((ROWS_PAD, WCNT), f32)

  sc_h0 = _sc_gather_scatter(18, D)
  sc_cnt = _sc_gather_scatter(80, WCNT)
  sc_layer = _sc_gather_scatter(80, D)

  h0_2 = sc_h0(node_tab, g_h0, s_h0, zeros_d)
  h = pl.pallas_call(
      _add2, out_shape=jax.ShapeDtypeStruct((N, D), f32),
  )(h0_2[0, :N], h0_2[1, :N])

  cnt2 = sc_cnt(jnp.asarray(oh_tab), g_cnt, s_edge, zeros_w)
  c0, c1 = cnt2[0, :N], cnt2[1, :N]

  n_layers = len(params["layers"])
  for l, lp in enumerate(params["layers"]):
    ep_l = jnp.zeros((WCNT, D), f32)
    ep_l = ep_l.at[0:7].set(lp["edge_emb"][0])
    ep_l = ep_l.at[7:10].set(lp["edge_emb"][1])
    ep_l = ep_l.at[10:13].set(lp["edge_emb"][2])
    ep_l = ep_l.at[13:19].set(lp["edge_emb"][3])
    agg2 = sc_layer(h, g_edge, s_edge, zeros_d)
    h = pl.pallas_call(
        _dense_update(l < n_layers - 1),
        out_shape=jax.ShapeDtypeStruct((N, D), f32),
    )(agg2[0, :N], agg2[1, :N], h, c0, c1,
      jnp.asarray(selfrow), ep_l,
      lp["W1"], lp["b1"].reshape(1, -1), lp["W2"], lp["b2"].reshape(1, -1),
      lp["gamma"].reshape(1, -1), lp["beta"].reshape(1, -1))
  return h


# speedup vs baseline: 1.0621x; 1.0621x over previous
import functools

import jax
import jax.numpy as jnp
import numpy as np
from jax import lax
from jax.experimental import pallas as pl
from jax.experimental.pallas import tpu as pltpu
from jax.experimental.pallas import tpu_sc as plsc

N = 10000
E = 320000
D = 128
NC = 2
NS = 16
NW = NC * NS
CHUNK = 128
ROWS_PAD = 10240
ROW_TRASH = N

ET = E + N
NCHUNK = 16
CH_ROWS = ET // NCHUNK
SEG = 32
NSEG = 6
CPT = SEG * NSEG
WINP = 6400
WIN = 6336
TRASH_W = 6336

NODE_SIZES = [121, 11, 11, 4, 7, 9, 7]
EDGE_SIZES = [7, 3, 3, 6]
NCOMBO = 7 * 3 * 3 * 6


def _sc_gather_scatter(chunks: int, d: int):
  mesh = plsc.VectorSubcoreMesh(core_axis_name="c", subcore_axis_name="s")
  rows_per_sub = ROWS_PAD // NS
  n_out_blk = rows_per_sub // CHUNK

  @functools.partial(
      pl.kernel,
      out_type=jax.ShapeDtypeStruct((NC, ROWS_PAD, d), jnp.float32),
      mesh=mesh,
      compiler_params=pltpu.CompilerParams(use_tc_tiling_on_sc=False),
      scratch_types=[
          pltpu.VMEM((chunks, CHUNK), jnp.int32),
          pltpu.VMEM((chunks, CHUNK), jnp.int32),
          pltpu.VMEM((CHUNK, d), jnp.float32),
          pltpu.VMEM_SHARED((ROWS_PAD, d), jnp.float32),
          pltpu.SemaphoreType.DMA,
      ],
  )
  def k(table_hbm, gidx_hbm, sidx_hbm, zeros_hbm, out_hbm,
        gi_v, si_v, rows_v, agg_sh, sem):
    c = lax.axis_index("c")
    s = lax.axis_index("s")
    wid = c * NS + s
    pltpu.sync_copy(gidx_hbm.at[pl.ds(wid * chunks, chunks)], gi_v)
    pltpu.sync_copy(sidx_hbm.at[pl.ds(wid * chunks, chunks)], si_v)
    pltpu.sync_copy(zeros_hbm.at[pl.ds(s * rows_per_sub, rows_per_sub)],
                    agg_sh.at[pl.ds(s * rows_per_sub, rows_per_sub)])
    plsc.subcore_barrier()

    def body(j, carry):
      pltpu.async_copy(table_hbm.at[gi_v.at[j]], rows_v, sem).wait()
      pltpu.sync_copy(rows_v, agg_sh.at[si_v.at[j]], add=True)
      return carry

    lax.fori_loop(0, chunks, body, 0, unroll=False)
    plsc.subcore_barrier()
    for b in range(n_out_blk):
      r0 = s * rows_per_sub + b * CHUNK
      pltpu.sync_copy(agg_sh.at[pl.ds(r0, CHUNK)], rows_v)
      pltpu.sync_copy(rows_v, out_hbm.at[c].at[pl.ds(r0, CHUNK)])

  return k


def _make_sc_layer():
  mesh = plsc.VectorSubcoreMesh(core_axis_name="c", subcore_axis_name="s")
  rows_per_sub = WINP // NS

  @functools.partial(
      pl.kernel,
      out_type=jax.ShapeDtypeStruct((NC, WINP, D), jnp.float32),
      mesh=mesh,
      compiler_params=pltpu.CompilerParams(use_tc_tiling_on_sc=False),
      scratch_types=[
          pltpu.VMEM((SEG, CHUNK), jnp.int32),
          pltpu.VMEM((SEG, CHUNK), jnp.int32),
          pltpu.VMEM((SEG, CHUNK), jnp.int32),
          pltpu.VMEM((SEG, CHUNK), jnp.int32),
          pltpu.VMEM((CHUNK, D), jnp.float32),
          pltpu.VMEM((CHUNK, D), jnp.float32),
          pltpu.VMEM_SHARED((WINP, D), jnp.float32),
          pltpu.SemaphoreType.DMA,
      ],
  )
  def k(h_hbm, tl_hbm, gidx_hbm, pkidx_hbm, zeros_hbm, out_hbm,
        gi_v, pk_v, ci_v, si_v, rows_v, emb_v, agg_sh, sem):
    c = lax.axis_index("c")
    s = lax.axis_index("s")
    chunk = c * 8 + s
    pltpu.sync_copy(zeros_hbm.at[pl.ds(s * rows_per_sub, rows_per_sub)],
                    agg_sh.at[pl.ds(s * rows_per_sub, rows_per_sub)])
    plsc.subcore_barrier()

    @pl.when(s < 8)
    def _():
      def seg_body(g, carry0):
        r0 = chunk * CPT + g * SEG
        pltpu.sync_copy(gidx_hbm.at[pl.ds(r0, SEG)], gi_v)
        pltpu.sync_copy(pkidx_hbm.at[pl.ds(r0, SEG)], pk_v)

        def unpack(i, carry2):
          for b in range(CHUNK // 16):
            sl = pl.ds(b * 16, 16)
            v = pk_v[i, sl]
            si_v[i, sl] = jax.lax.shift_right_logical(v, 9)
            ci_v[i, sl] = jax.lax.bitwise_and(v, 511)
          return carry2

        lax.fori_loop(0, SEG, unpack, 0, unroll=False)

        def body(j, carry):
          pltpu.async_copy(h_hbm.at[gi_v.at[j]], rows_v, sem).wait()
          pltpu.async_copy(tl_hbm.at[ci_v.at[j]], emb_v, sem).wait()

          def add_row(i, carry2):
            for b in range(D // 16):
              sl = pl.ds(b * 16, 16)
              rows_v[i, sl] = rows_v[i, sl] + emb_v[i, sl]
            return carry2

          lax.fori_loop(0, CHUNK, add_row, 0, unroll=False)
          pltpu.sync_copy(rows_v, agg_sh.at[si_v.at[j]], add=True)
          return carry

        lax.fori_loop(0, SEG, body, 0, unroll=False)
        return carry0

      lax.fori_loop(0, NSEG, seg_body, 0, unroll=False)
    plsc.subcore_barrier()
    for b in range(rows_per_sub // 80):
      r0 = s * rows_per_sub + b * 80
      pltpu.sync_copy(agg_sh.at[pl.ds(r0, 80)], rows_v.at[pl.ds(0, 80)])
      pltpu.sync_copy(rows_v.at[pl.ds(0, 80)], out_hbm.at[c].at[pl.ds(r0, 80)])

  return k


def _mlp(agg_r, w1_r, b1_r, w2_r, b2_r, out_r):
  z1 = jnp.maximum(
      jnp.dot(agg_r[...], w1_r[...], preferred_element_type=jnp.float32)
      + b1_r[...], 0.0)
  out_r[...] = jnp.dot(z1, w2_r[...],
                       preferred_element_type=jnp.float32) + b2_r[...]


def _add2(a0, a1, out):
  out[...] = a0[...] + a1[...]


def _pad_idx(a, pad_val, total):
  a = a.astype(jnp.int32)
  return jnp.concatenate(
      [a, jnp.full((total - a.shape[0],), pad_val, jnp.int32)]
  ).reshape(-1, CHUNK)


def _chunk_pad(a, pad_val):
  a = a.astype(jnp.int32).reshape(NCHUNK, CH_ROWS)
  a = jnp.pad(a, ((0, 0), (0, CPT * CHUNK - CH_ROWS)),
              constant_values=pad_val)
  return a.reshape(-1, CHUNK)


def kernel(x, edge_index, edge_attr, params):
  f32 = jnp.float32
  src = edge_index[0].astype(jnp.int32)
  dst = edge_index[1].astype(jnp.int32)
  ea = edge_attr.astype(jnp.int32)

  combo = ((ea[:, 0] * 3 + ea[:, 1]) * 3 + ea[:, 2]) * 6 + ea[:, 3]
  loop = jnp.arange(N, dtype=jnp.int32)
  src_f = jnp.concatenate([src, loop])
  dst_f = jnp.concatenate([dst, loop])
  combo_f = jnp.concatenate([combo, jnp.full((N,), 216, jnp.int32)])

  perm = jnp.argsort(dst_f, stable=True).astype(jnp.int32)
  sd = dst_f[perm]
  gs = src_f[perm]
  gc = combo_f[perm]

  b1 = sd[8 * CH_ROWS]
  offs = jnp.concatenate([jnp.zeros((8 * CH_ROWS,), jnp.int32),
                          jnp.full((8 * CH_ROWS,), b1, jnp.int32)])
  sloc = jnp.clip(sd - offs, 0, TRASH_W)

  g_edge = _chunk_pad(gs, 0)
  pk_edge = _chunk_pad(sloc * 512 + gc, TRASH_W * 512 + NCOMBO)

  cids = np.arange(NCOMBO)
  i0, r = np.divmod(cids, 54)
  i1, r = np.divmod(r, 18)
  i2, i3 = np.divmod(r, 6)

  node_tab = jnp.concatenate(list(params["node_emb"]), axis=0)
  noffs = jnp.asarray(np.cumsum([0] + NODE_SIZES[:-1]), jnp.int32)
  h0_g = (x.astype(jnp.int32) + noffs[None, :]).reshape(-1)
  h0_s = jnp.repeat(jnp.arange(N, dtype=jnp.int32), 7)
  ep_h0 = NW * 24 * CHUNK
  g_h0 = _pad_idx(h0_g, 0, ep_h0)
  s_h0 = _pad_idx(h0_s, ROW_TRASH, ep_h0)

  zeros_d = jnp.zeros((ROWS_PAD, D), f32)
  zeros_w = jnp.zeros((WINP, D), f32)

  sc_h0 = _sc_gather_scatter(24, D)
  sc_layer = _make_sc_layer()

  h0_2 = sc_h0(node_tab, g_h0, s_h0, zeros_d)
  h = pl.pallas_call(
      _add2, out_shape=jax.ShapeDtypeStruct((N, D), f32),
  )(h0_2[0, :N], h0_2[1, :N])

  n_layers = len(params["layers"])
  for l, lp in enumerate(params["layers"]):
    ee = lp["edge_emb"]
    tl = ((ee[0][i0] + ee[1][i1]) + ee[2][i2]) + ee[3][i3]
    tl = jnp.concatenate([tl, jnp.zeros((6, D), f32)], axis=0)

    out2 = sc_layer(h, tl, g_edge, pk_edge, zeros_w)
    tall = N + WIN
    agg = (lax.dynamic_update_slice(jnp.zeros((tall, D), f32),
                                    out2[0, :WIN], (0, 0))
           + lax.dynamic_update_slice(jnp.zeros((tall, D), f32),
                                      out2[1, :WIN], (b1, 0)))[:N]

    z = pl.pallas_call(
        _mlp, out_shape=jax.ShapeDtypeStruct((N, D), f32),
    )(agg, lp["W1"], lp["b1"].reshape(1, -1),
      lp["W2"], lp["b2"].reshape(1, -1))

    mean = jnp.mean(z, axis=0)
    var = jnp.var(z, axis=0)
    z = (z - mean) / jnp.sqrt(var + 1e-5) * lp["gamma"] + lp["beta"]
    if l < n_layers - 1:
      z = jnp.where(z > 0, z, jnp.expm1(z))
    h = z
  return h
